# baseline (speedup 1.0000x reference)
import functools

import jax
import jax.numpy as jnp
from jax.experimental import pallas as pl
from jax.experimental.pallas import tpu as pltpu
from jax.experimental.pallas import tpu_sc as plsc

_KBLK = 2048
_GATHER_WINDOW = 128


def _tc_body(nsteps, rem, post_ref, yt_ref, acc_ref, cnt_ref):
    k = pl.program_id(0)

    @pl.when(k == 0)
    def _init():
        acc_ref[...] = jnp.zeros_like(acc_ref)
        cnt_ref[...] = jnp.zeros_like(cnt_ref)

    maskf = jnp.where(post_ref[...] > 0, 1.0, 0.0)
    yf = yt_ref[...]

    def _tail(ops):
        m, y = ops
        kblk, b = m.shape
        d = y.shape[0]
        row = jax.lax.broadcasted_iota(jnp.int32, (kblk, b), 0)
        col = jax.lax.broadcasted_iota(jnp.int32, (d, kblk), 1)
        return jnp.where(row < rem, m, 0.0), jnp.where(col < rem, y, 0.0)

    maskf, yf = jax.lax.cond(k == nsteps - 1, _tail, lambda o: o,
                             (maskf, yf))
    cnt_ref[...] += jnp.sum(maskf, axis=0, keepdims=True)
    acc_ref[...] += jax.lax.dot(yf.astype(jnp.bfloat16),
                                maskf.astype(jnp.bfloat16),
                                preferred_element_type=jnp.float32)


def _tc_call(post, yt):
    n, b = post.shape
    d = yt.shape[0]
    nsteps = pl.cdiv(n, _KBLK)
    rem = n - (nsteps - 1) * _KBLK
    return pl.pallas_call(
        functools.partial(_tc_body, nsteps, rem),
        grid=(nsteps,),
        in_specs=[
            pl.BlockSpec((_KBLK, b), lambda k: (k, 0)),
            pl.BlockSpec((d, _KBLK), lambda k: (0, k)),
        ],
        out_specs=[pl.BlockSpec((d, b), lambda k: (0, 0)),
                   pl.BlockSpec((1, b), lambda k: (0, 0))],
        out_shape=[jax.ShapeDtypeStruct((d, b), jnp.float32),
                   jax.ShapeDtypeStruct((1, b), jnp.float32)],
        compiler_params=pltpu.CompilerParams(
            dimension_semantics=("arbitrary",)),
    )(post, yt)


def _combine_body(acc_ref, cnt_ref, uet_ref, iet_ref, ub_ref, ib_ref,
                  gb_ref, out_ref):
    puyj = acc_ref[...] / cnt_ref[...]
    dot = jnp.sum((puyj + uet_ref[...]) * iet_ref[...], axis=0,
                  keepdims=True)
    out_ref[...] = dot + ub_ref[...] + ib_ref[...] + gb_ref[0, 0]


def _combine_call(acc, cnt, uet, iet, ube, ibe, gb2d):
    b = acc.shape[1]
    return pl.pallas_call(
        _combine_body,
        out_shape=jax.ShapeDtypeStruct((1, b), jnp.float32),
    )(acc, cnt, uet, iet, ube, ibe, gb2d)


def _sc_gather(user, item, UI, ub1d, ib1d):
    b = user.shape[0]
    d2 = UI.shape[1]
    mesh = plsc.VectorSubcoreMesh(core_axis_name="c", subcore_axis_name="s")
    nw = mesh.num_cores * mesh.num_subcores
    bw = b // nw
    out_types = (
        jax.ShapeDtypeStruct((b, d2), jnp.float32),
        jax.ShapeDtypeStruct((b, d2), jnp.float32),
        jax.ShapeDtypeStruct((b,), jnp.float32),
        jax.ShapeDtypeStruct((b,), jnp.float32),
    )

    @functools.partial(
        pl.kernel, mesh=mesh, out_type=out_types,
        scratch_types=[
            pltpu.VMEM((bw,), jnp.int32),
            pltpu.VMEM((bw,), jnp.int32),
            pltpu.VMEM((bw, d2), jnp.float32),
            pltpu.VMEM((bw, d2), jnp.float32),
            pltpu.VMEM((bw,), jnp.float32),
            pltpu.VMEM((bw,), jnp.float32),
            pltpu.SemaphoreType.DMA,
        ])
    def sc_kernel(tab_hbm, ub_hbm, ib_hbm, ui_hbm, ii_hbm,
                  ue_hbm, ie_hbm, ube_hbm, ibe_hbm,
                  uidx_v, iidx_v, ue_v, ie_v, ub_v, ib_v, sem):
        wid = (jax.lax.axis_index("s") * mesh.num_cores
               + jax.lax.axis_index("c"))
        base = wid * bw
        pltpu.sync_copy(ui_hbm.at[pl.ds(base, bw)], uidx_v)
        pltpu.sync_copy(ii_hbm.at[pl.ds(base, bw)], iidx_v)
        c1 = pltpu.async_copy(tab_hbm.at[uidx_v], ue_v, sem)
        c2 = pltpu.async_copy(tab_hbm.at[iidx_v], ie_v, sem)
        c3 = pltpu.async_copy(ub_hbm.at[uidx_v], ub_v, sem)
        c4 = pltpu.async_copy(ib_hbm.at[iidx_v], ib_v, sem)
        c1.wait()
        c2.wait()
        c3.wait()
        c4.wait()
        pltpu.sync_copy(ue_v, ue_hbm.at[pl.ds(base, bw)])
        pltpu.sync_copy(ie_v, ie_hbm.at[pl.ds(base, bw)])
        pltpu.sync_copy(ub_v, ube_hbm.at[pl.ds(base, bw)])
        pltpu.sync_copy(ib_v, ibe_hbm.at[pl.ds(base, bw)])

    return sc_kernel(UI, ub1d, ib1d, user, item)


def kernel(user, item, pos, U_MF, I_MF, Y, U_BIAS, I_BIAS, GB):
    b, n = pos.shape
    d = Y.shape[1]
    UI = jnp.concatenate([U_MF, I_MF], axis=1)
    ue, ie, ube, ibe = _sc_gather(user, item, UI,
                                  U_BIAS.reshape(-1), I_BIAS.reshape(-1))
    acc, cnt = _tc_call(pos.T, Y.T)
    uet = ue[:, :d].T
    iet = ie[:, d:].T
    out2d = _combine_call(acc, cnt, uet, iet, ube.reshape(1, b),
                          ibe.reshape(1, b), GB.reshape(1, 1))
    return out2d.reshape(b)


# speedup vs baseline: 1.4053x; 1.4053x over previous
import functools

import jax
import jax.numpy as jnp
from jax.experimental import pallas as pl
from jax.experimental.pallas import tpu as pltpu
from jax.experimental.pallas import tpu_sc as plsc

_KBLK = 2048
_GATHER_WINDOW = 128


def _tc_body(nsteps, rem, post_ref, yt_ref, acc_ref):
    k = pl.program_id(0)
    d, kblk = yt_ref.shape

    @pl.when(k == 0)
    def _init():
        acc_ref[...] = jnp.zeros_like(acc_ref)

    klimit = jnp.where(k == nsteps - 1, rem, kblk)
    mbf = jnp.where(post_ref[...] > 0, 1.0, 0.0).astype(jnp.bfloat16)
    lane_y = jax.lax.broadcasted_iota(jnp.int32, (d, kblk), 1)
    yz = jnp.where(lane_y < klimit, yt_ref[...], 0.0)
    sub = jax.lax.broadcasted_iota(jnp.int32, (16, kblk), 0)
    lane_o = jax.lax.broadcasted_iota(jnp.int32, (16, kblk), 1)
    ones16 = jnp.where((sub == 0) & (lane_o < klimit), 1.0, 0.0)
    yplus = jnp.concatenate([yz, ones16], axis=0).astype(jnp.bfloat16)
    acc_ref[...] += jax.lax.dot(yplus, mbf,
                                preferred_element_type=jnp.float32)


def _tc_call(post, yt):
    n, b = post.shape
    d = yt.shape[0]
    nsteps = pl.cdiv(n, _KBLK)
    rem = n - (nsteps - 1) * _KBLK
    return pl.pallas_call(
        functools.partial(_tc_body, nsteps, rem),
        grid=(nsteps,),
        in_specs=[
            pl.BlockSpec((_KBLK, b), lambda k: (k, 0)),
            pl.BlockSpec((d, _KBLK), lambda k: (0, k)),
        ],
        out_specs=pl.BlockSpec((d + 16, b), lambda k: (0, 0)),
        out_shape=jax.ShapeDtypeStruct((d + 16, b), jnp.float32),
        compiler_params=pltpu.CompilerParams(
            dimension_semantics=("arbitrary",)),
    )(post, yt)


def _combine_body(d, acc_ref, uet_ref, iet_ref, ub_ref, ib_ref,
                  gb_ref, out_ref):
    acc = acc_ref[...]
    cnt = acc[d:d + 1, :]
    puyj = acc[:d, :] / cnt
    dot = jnp.sum((puyj + uet_ref[...]) * iet_ref[...], axis=0,
                  keepdims=True)
    out_ref[...] = dot + ub_ref[...] + ib_ref[...] + gb_ref[0, 0]


def _combine_call(acc, uet, iet, ube, ibe, gb2d, d):
    b = acc.shape[1]
    return pl.pallas_call(
        functools.partial(_combine_body, d),
        out_shape=jax.ShapeDtypeStruct((1, b), jnp.float32),
    )(acc, uet, iet, ube, ibe, gb2d)


def _sc_gather(user, item, UI, ub1d, ib1d):
    b = user.shape[0]
    d2 = UI.shape[1]
    mesh = plsc.VectorSubcoreMesh(core_axis_name="c", subcore_axis_name="s")
    nw = mesh.num_cores * mesh.num_subcores
    bw = b // nw
    out_types = (
        jax.ShapeDtypeStruct((b, d2), jnp.float32),
        jax.ShapeDtypeStruct((b, d2), jnp.float32),
        jax.ShapeDtypeStruct((b,), jnp.float32),
        jax.ShapeDtypeStruct((b,), jnp.float32),
    )

    @functools.partial(
        pl.kernel, mesh=mesh, out_type=out_types,
        scratch_types=[
            pltpu.VMEM((bw,), jnp.int32),
            pltpu.VMEM((bw,), jnp.int32),
            pltpu.VMEM((bw, d2), jnp.float32),
            pltpu.VMEM((bw, d2), jnp.float32),
            pltpu.VMEM((bw,), jnp.float32),
            pltpu.VMEM((bw,), jnp.float32),
            pltpu.SemaphoreType.DMA,
        ])
    def sc_kernel(tab_hbm, ub_hbm, ib_hbm, ui_hbm, ii_hbm,
                  ue_hbm, ie_hbm, ube_hbm, ibe_hbm,
                  uidx_v, iidx_v, ue_v, ie_v, ub_v, ib_v, sem):
        wid = (jax.lax.axis_index("s") * mesh.num_cores
               + jax.lax.axis_index("c"))
        base = wid * bw
        pltpu.sync_copy(ui_hbm.at[pl.ds(base, bw)], uidx_v)
        pltpu.sync_copy(ii_hbm.at[pl.ds(base, bw)], iidx_v)
        c1 = pltpu.async_copy(tab_hbm.at[uidx_v], ue_v, sem)
        c2 = pltpu.async_copy(tab_hbm.at[iidx_v], ie_v, sem)
        c3 = pltpu.async_copy(ub_hbm.at[uidx_v], ub_v, sem)
        c4 = pltpu.async_copy(ib_hbm.at[iidx_v], ib_v, sem)
        c1.wait()
        c2.wait()
        c3.wait()
        c4.wait()
        pltpu.sync_copy(ue_v, ue_hbm.at[pl.ds(base, bw)])
        pltpu.sync_copy(ie_v, ie_hbm.at[pl.ds(base, bw)])
        pltpu.sync_copy(ub_v, ube_hbm.at[pl.ds(base, bw)])
        pltpu.sync_copy(ib_v, ibe_hbm.at[pl.ds(base, bw)])

    return sc_kernel(UI, ub1d, ib1d, user, item)


def kernel(user, item, pos, U_MF, I_MF, Y, U_BIAS, I_BIAS, GB):
    b, n = pos.shape
    d = Y.shape[1]
    UI = jnp.concatenate([U_MF, I_MF], axis=1)
    ue, ie, ube, ibe = _sc_gather(user, item, UI,
                                  U_BIAS.reshape(-1), I_BIAS.reshape(-1))
    acc = _tc_call(pos.T, Y.T)
    uet = ue[:, :d].T
    iet = ie[:, d:].T
    out2d = _combine_call(acc, uet, iet, ube.reshape(1, b),
                          ibe.reshape(1, b), GB.reshape(1, 1), d)
    return out2d.reshape(b)
